# baseline (speedup 1.0000x reference)
import functools

import jax
import jax.numpy as jnp
from jax import lax
from jax.experimental import pallas as pl
from jax.experimental.pallas import tpu as pltpu
from jax.experimental.pallas import tpu_sc as plsc

D, H, W = 160, 192, 224
N = D * H * W
HW = H * W
NC, NS = 2, 16
NW = NC * NS

BZ, BY = 5, 6
HALO = 6
SNZ, SNY = BZ + 2 * HALO + 1, BY + 2 * HALO + 1
SNYW = SNY * W
SLABW = SNZ * SNYW
NB_Z, NB_Y = D // BZ, H // BY
BLKV = BZ * BY * W
ROWV = BY * W
VPR = W // 16


def _floor(c):
    t = c.astype(jnp.int32)
    tf = t.astype(jnp.float32)
    adj = tf > c
    fi = t - jnp.where(adj, 1, 0)
    ff = tf - jnp.where(adj, 1.0, 0.0)
    return fi, c - ff


def _dim(c, size):
    fi, fr = _floor(c)
    fi1 = fi + 1
    c0 = jnp.clip(fi, 0, size - 1)
    c1 = jnp.clip(fi1, 0, size - 1)
    m0 = fi.astype(jnp.uint32) < jnp.uint32(size)
    m1 = fi1.astype(jnp.uint32) < jnp.uint32(size)
    w0 = jnp.where(m0, 1.0 - fr, 0.0)
    w1 = jnp.where(m1, fr, 0.0)
    return c0, c1, w0, w1


def _corners(cz, cy, cx, zf_shift, yf_shift):
    z0, z1, wz0, wz1 = _dim(cz, D)
    y0, y1, wy0, wy1 = _dim(cy, H)
    x0, x1, wx0, wx1 = _dim(cx, W)
    zl0 = z0 - zf_shift
    zl1 = z1 - zf_shift
    yl0 = y0 - yf_shift
    yl1 = y1 - yf_shift
    wzy = (wz0 * wy0, wz0 * wy1, wz1 * wy0, wz1 * wy1)
    return (zl0, zl1, yl0, yl1), (x0, x1), wzy, (wx0, wx1)


def _body(src_hbm, flow_hbm, out_hbm, slab, flz, fly, flx, ob, fb,
          sdma, sout, sfb):
    wid = lax.axis_index("s") * NC + lax.axis_index("c")
    iota_f = lax.iota(jnp.int32, 16).astype(jnp.float32)
    y0b = wid * BY
    sylo = jnp.clip(y0b - HALO, 0, H - SNY)

    def blk_body(blk, prev_end):
        z0b = blk * BZ
        szlo = jnp.clip(z0b - HALO, 0, D - SNZ)
        bs = szlo % SNZ

        conds = []
        for i in range(SNZ):
            zg = szlo + i
            cond = zg >= prev_end
            conds.append(cond)

            @pl.when(cond)
            def _(zg=zg):
                slot = zg % SNZ
                off = (zg * H + sylo) * W
                pltpu.async_copy(src_hbm.at[pl.ds(off, SNYW)],
                                 slab.at[pl.ds(slot * SNYW, SNYW)], sdma)

        for zz in range(BZ):
            off = ((z0b + zz) * H + y0b) * W
            pltpu.async_copy(flow_hbm.at[pl.ds(off, ROWV)],
                             flz.at[pl.ds(zz * ROWV, ROWV)], sdma)
            pltpu.async_copy(flow_hbm.at[pl.ds(N + off, ROWV)],
                             fly.at[pl.ds(zz * ROWV, ROWV)], sdma)
            pltpu.async_copy(flow_hbm.at[pl.ds(2 * N + off, ROWV)],
                             flx.at[pl.ds(zz * ROWV, ROWV)], sdma)

        @pl.when(blk > 0)
        def _():
            for _ in range(BZ):
                pltpu.make_async_copy(
                    ob.at[pl.ds(0, ROWV)],
                    out_hbm.at[pl.ds(0, ROWV)], sout).wait()

        for i in range(SNZ):
            @pl.when(conds[i])
            def _():
                pltpu.make_async_copy(src_hbm.at[pl.ds(0, SNYW)],
                                      slab.at[pl.ds(0, SNYW)], sdma).wait()
        for _ in range(BZ * 3):
            pltpu.make_async_copy(flow_hbm.at[pl.ds(0, ROWV)],
                                  flz.at[pl.ds(0, ROWV)], sdma).wait()

        def row(rr, c2):
            zz = rr // BY
            yy = rr % BY
            zf = (z0b + zz).astype(jnp.float32)
            yf = (y0b + yy).astype(jnp.float32)
            o0 = rr * W

            def vec(v, okacc):
                dsl = pl.ds(o0 + v * 16, 16)
                xv = iota_f + (v * 16).astype(jnp.float32)
                cz = flz[dsl] + zf
                cy = fly[dsl] + yf
                cx = flx[dsl] + xv
                (zl0, zl1, yl0, yl1), (x0, x1), wzy, (wx0, wx1) = _corners(
                    cz, cy, cx, szlo, sylo)
                inz0 = zl0.astype(jnp.uint32) < jnp.uint32(SNZ)
                inz1 = zl1.astype(jnp.uint32) < jnp.uint32(SNZ)
                iny0 = yl0.astype(jnp.uint32) < jnp.uint32(SNY)
                iny1 = yl1.astype(jnp.uint32) < jnp.uint32(SNY)
                s0 = zl0 + bs
                s0 = s0 - jnp.where(s0 >= SNZ, SNZ, 0)
                s1 = zl1 + bs
                s1 = s1 - jnp.where(s1 >= SNZ, SNZ, 0)
                rb00 = s0 * SNYW + yl0 * W
                rb01 = s0 * SNYW + yl1 * W
                rb10 = s1 * SNYW + yl0 * W
                rb11 = s1 * SNYW + yl1 * W
                rbs = (rb00, rb01, rb10, rb11)
                ins = (inz0 & iny0, inz0 & iny1, inz1 & iny0, inz1 & iny1)
                zero = jnp.zeros((16,), jnp.int32)
                acc = None
                for q in range(4):
                    lq0 = jnp.where(ins[q], rbs[q] + x0, zero)
                    lq1 = jnp.where(ins[q], rbs[q] + x1, zero)
                    t = (wx0 * plsc.load_gather(slab, [lq0]) +
                         wx1 * plsc.load_gather(slab, [lq1]))
                    acc = wzy[q] * t if acc is None else acc + wzy[q] * t
                ob[dsl] = acc
                ok4 = ins[0] & ins[1] & ins[2] & ins[3]
                return okacc & jnp.where(ok4, 1, zero)

            okv = plsc.parallel_loop(0, VPR, unroll=2, carry=jnp.ones((16,), jnp.int32))(vec)

            @pl.when(jnp.any(okv == 0))
            def _():
                def fvec(v, c3):
                    dsl = pl.ds(o0 + v * 16, 16)
                    xv = iota_f + (v * 16).astype(jnp.float32)
                    cz = flz[dsl] + zf
                    cy = fly[dsl] + yf
                    cx = flx[dsl] + xv
                    (gz0, gz1, gy0, gy1), (x0, x1), wzy, (wx0, wx1) = _corners(
                        cz, cy, cx, 0, 0)
                    wv = (wzy[0] * wx0, wzy[0] * wx1, wzy[1] * wx0,
                          wzy[1] * wx1, wzy[2] * wx0, wzy[2] * wx1,
                          wzy[3] * wx0, wzy[3] * wx1)
                    gbs = (gz0 * HW + gy0 * W, gz0 * HW + gy1 * W,
                           gz1 * HW + gy0 * W, gz1 * HW + gy1 * W)
                    cps = []
                    for q in range(4):
                        cps.append(pltpu.async_copy(
                            src_hbm.at[gbs[q] + x0], fb[2 * q], sfb))
                        cps.append(pltpu.async_copy(
                            src_hbm.at[gbs[q] + x1], fb[2 * q + 1], sfb))
                    for cp in cps:
                        cp.wait()
                    acc2 = wv[0] * fb[0][...]
                    for c in range(1, 8):
                        acc2 = acc2 + wv[c] * fb[c][...]
                    ob[dsl] = acc2
                    return c3

                lax.fori_loop(0, VPR, fvec, 0)

            return c2

        lax.fori_loop(0, BZ * BY, row, 0)

        for zz in range(BZ):
            off = ((z0b + zz) * H + y0b) * W
            pltpu.async_copy(ob.at[pl.ds(zz * ROWV, ROWV)],
                             out_hbm.at[pl.ds(off, ROWV)], sout)
        return szlo + SNZ

    lax.fori_loop(0, NB_Z, blk_body, jnp.int32(0))
    for _ in range(BZ):
        pltpu.make_async_copy(ob.at[pl.ds(0, ROWV)],
                              out_hbm.at[pl.ds(0, ROWV)], sout).wait()


@jax.jit
def _run(src_flat, flow_flat):
    mesh = plsc.VectorSubcoreMesh(core_axis_name="c", subcore_axis_name="s")
    f = functools.partial(
        pl.kernel,
        out_type=jax.ShapeDtypeStruct((N,), jnp.float32),
        mesh=mesh,
        compiler_params=pltpu.CompilerParams(needs_layout_passes=False),
        scratch_types=[
            pltpu.VMEM((SLABW,), jnp.float32),
            pltpu.VMEM((BLKV,), jnp.float32),
            pltpu.VMEM((BLKV,), jnp.float32),
            pltpu.VMEM((BLKV,), jnp.float32),
            pltpu.VMEM((BLKV,), jnp.float32),
            [pltpu.VMEM((16,), jnp.float32) for _ in range(8)],
            pltpu.SemaphoreType.DMA,
            pltpu.SemaphoreType.DMA,
            pltpu.SemaphoreType.DMA,
        ],
    )(_body)
    return f(src_flat, flow_flat)


def kernel(source, flow_field):
    src_flat = source.reshape(N)
    flow_flat = flow_field.reshape(3 * N)
    out = _run(src_flat, flow_flat)
    return out.reshape(source.shape)


# speedup vs baseline: 1.0093x; 1.0093x over previous
import functools

import jax
import jax.numpy as jnp
from jax import lax
from jax.experimental import pallas as pl
from jax.experimental.pallas import tpu as pltpu
from jax.experimental.pallas import tpu_sc as plsc

D, H, W = 160, 192, 224
N = D * H * W
HW = H * W
NC, NS = 2, 16
NW = NC * NS

BZ, BY = 5, 6
HALO = 6
SNZ, SNY = BZ + 2 * HALO + 1, BY + 2 * HALO + 1
SNYW = SNY * W
SLABW = SNZ * SNYW
NB_Z, NB_Y = D // BZ, H // BY
BLKV = BZ * BY * W
ROWV = BY * W
VPR = W // 16


def _floor(c):
    t = c.astype(jnp.int32)
    tf = t.astype(jnp.float32)
    adj = tf > c
    fi = t - jnp.where(adj, 1, 0)
    ff = tf - jnp.where(adj, 1.0, 0.0)
    return fi, c - ff


def _dim(c, size):
    fi, fr = _floor(c)
    fi1 = fi + 1
    c0 = jnp.clip(fi, 0, size - 1)
    c1 = jnp.clip(fi1, 0, size - 1)
    m0 = fi.astype(jnp.uint32) < jnp.uint32(size)
    m1 = fi1.astype(jnp.uint32) < jnp.uint32(size)
    w0 = jnp.where(m0, 1.0 - fr, 0.0)
    w1 = jnp.where(m1, fr, 0.0)
    return c0, c1, w0, w1


def _corners(cz, cy, cx, zf_shift, yf_shift):
    z0, z1, wz0, wz1 = _dim(cz, D)
    y0, y1, wy0, wy1 = _dim(cy, H)
    x0, x1, wx0, wx1 = _dim(cx, W)
    zl0 = z0 - zf_shift
    zl1 = z1 - zf_shift
    yl0 = y0 - yf_shift
    yl1 = y1 - yf_shift
    w00 = wz0 * wy0
    w01 = wz0 * wy1
    w10 = wz1 * wy0
    w11 = wz1 * wy1
    wv = (w00 * wx0, w00 * wx1, w01 * wx0, w01 * wx1,
          w10 * wx0, w10 * wx1, w11 * wx0, w11 * wx1)
    return (zl0, zl1, yl0, yl1), (x0, x1), wv


def _body(src_hbm, flow_hbm, out_hbm, slab, flz, fly, flx, ob, fb,
          sdma, sout, sfb):
    wid = lax.axis_index("s") * NC + lax.axis_index("c")
    iota_f = lax.iota(jnp.int32, 16).astype(jnp.float32)
    y0b = wid * BY
    sylo = jnp.clip(y0b - HALO, 0, H - SNY)

    def blk_body(blk, prev_end):
        z0b = blk * BZ
        szlo = jnp.clip(z0b - HALO, 0, D - SNZ)
        bs = szlo % SNZ

        conds = []
        for i in range(SNZ):
            zg = szlo + i
            cond = zg >= prev_end
            conds.append(cond)

            @pl.when(cond)
            def _(zg=zg):
                slot = zg % SNZ
                off = (zg * H + sylo) * W
                pltpu.async_copy(src_hbm.at[pl.ds(off, SNYW)],
                                 slab.at[pl.ds(slot * SNYW, SNYW)], sdma)

        for zz in range(BZ):
            off = ((z0b + zz) * H + y0b) * W
            pltpu.async_copy(flow_hbm.at[pl.ds(off, ROWV)],
                             flz.at[pl.ds(zz * ROWV, ROWV)], sdma)
            pltpu.async_copy(flow_hbm.at[pl.ds(N + off, ROWV)],
                             fly.at[pl.ds(zz * ROWV, ROWV)], sdma)
            pltpu.async_copy(flow_hbm.at[pl.ds(2 * N + off, ROWV)],
                             flx.at[pl.ds(zz * ROWV, ROWV)], sdma)

        @pl.when(blk > 0)
        def _():
            for _ in range(BZ):
                pltpu.make_async_copy(
                    ob.at[pl.ds(0, ROWV)],
                    out_hbm.at[pl.ds(0, ROWV)], sout).wait()

        for i in range(SNZ):
            @pl.when(conds[i])
            def _():
                pltpu.make_async_copy(src_hbm.at[pl.ds(0, SNYW)],
                                      slab.at[pl.ds(0, SNYW)], sdma).wait()
        for _ in range(BZ * 3):
            pltpu.make_async_copy(flow_hbm.at[pl.ds(0, ROWV)],
                                  flz.at[pl.ds(0, ROWV)], sdma).wait()

        def row(rr, c2):
            zz = rr // BY
            yy = rr % BY
            zf = (z0b + zz).astype(jnp.float32)
            yf = (y0b + yy).astype(jnp.float32)
            o0 = rr * W

            def vec(v, okacc):
                dsl = pl.ds(o0 + v * 16, 16)
                xv = iota_f + (v * 16).astype(jnp.float32)
                cz = flz[dsl] + zf
                cy = fly[dsl] + yf
                cx = flx[dsl] + xv
                (zl0, zl1, yl0, yl1), (x0, x1), wv = _corners(
                    cz, cy, cx, szlo, sylo)
                inz0 = zl0.astype(jnp.uint32) < jnp.uint32(SNZ)
                inz1 = zl1.astype(jnp.uint32) < jnp.uint32(SNZ)
                iny0 = yl0.astype(jnp.uint32) < jnp.uint32(SNY)
                iny1 = yl1.astype(jnp.uint32) < jnp.uint32(SNY)
                s0 = zl0 + bs
                s0 = s0 - jnp.where(s0 >= SNZ, SNZ, 0)
                s1 = zl1 + bs
                s1 = s1 - jnp.where(s1 >= SNZ, SNZ, 0)
                rb00 = s0 * SNYW + yl0 * W
                rb01 = s0 * SNYW + yl1 * W
                rb10 = s1 * SNYW + yl0 * W
                rb11 = s1 * SNYW + yl1 * W
                rbs = (rb00, rb01, rb10, rb11)
                ins = (inz0 & iny0, inz0 & iny1, inz1 & iny0, inz1 & iny1)
                zero = jnp.zeros((16,), jnp.int32)
                acc = None
                for q in range(4):
                    lq0 = jnp.where(ins[q], rbs[q] + x0, zero)
                    lq1 = jnp.where(ins[q], rbs[q] + x1, zero)
                    t = (wv[2 * q] * plsc.load_gather(slab, [lq0]) +
                         wv[2 * q + 1] * plsc.load_gather(slab, [lq1]))
                    acc = t if acc is None else acc + t
                ob[dsl] = acc
                ok4 = ins[0] & ins[1] & ins[2] & ins[3]
                return okacc & jnp.where(ok4, 1, zero)

            okv = plsc.parallel_loop(0, VPR, carry=jnp.ones((16,), jnp.int32))(vec)

            @pl.when(jnp.any(okv == 0))
            def _():
                def fvec(v, c3):
                    dsl = pl.ds(o0 + v * 16, 16)
                    xv = iota_f + (v * 16).astype(jnp.float32)
                    cz = flz[dsl] + zf
                    cy = fly[dsl] + yf
                    cx = flx[dsl] + xv
                    (gz0, gz1, gy0, gy1), (x0, x1), wv = _corners(
                        cz, cy, cx, 0, 0)
                    gbs = (gz0 * HW + gy0 * W, gz0 * HW + gy1 * W,
                           gz1 * HW + gy0 * W, gz1 * HW + gy1 * W)
                    cps = []
                    for q in range(4):
                        cps.append(pltpu.async_copy(
                            src_hbm.at[gbs[q] + x0], fb[2 * q], sfb))
                        cps.append(pltpu.async_copy(
                            src_hbm.at[gbs[q] + x1], fb[2 * q + 1], sfb))
                    for cp in cps:
                        cp.wait()
                    acc2 = wv[0] * fb[0][...]
                    for c in range(1, 8):
                        acc2 = acc2 + wv[c] * fb[c][...]
                    ob[dsl] = acc2
                    return c3

                lax.fori_loop(0, VPR, fvec, 0)

            return c2

        lax.fori_loop(0, BZ * BY, row, 0)

        for zz in range(BZ):
            off = ((z0b + zz) * H + y0b) * W
            pltpu.async_copy(ob.at[pl.ds(zz * ROWV, ROWV)],
                             out_hbm.at[pl.ds(off, ROWV)], sout)
        return szlo + SNZ

    lax.fori_loop(0, NB_Z, blk_body, jnp.int32(0))
    for _ in range(BZ):
        pltpu.make_async_copy(ob.at[pl.ds(0, ROWV)],
                              out_hbm.at[pl.ds(0, ROWV)], sout).wait()


@jax.jit
def _run(src_flat, flow_flat):
    mesh = plsc.VectorSubcoreMesh(core_axis_name="c", subcore_axis_name="s")
    f = functools.partial(
        pl.kernel,
        out_type=jax.ShapeDtypeStruct((N,), jnp.float32),
        mesh=mesh,
        compiler_params=pltpu.CompilerParams(needs_layout_passes=False),
        scratch_types=[
            pltpu.VMEM((SLABW,), jnp.float32),
            pltpu.VMEM((BLKV,), jnp.float32),
            pltpu.VMEM((BLKV,), jnp.float32),
            pltpu.VMEM((BLKV,), jnp.float32),
            pltpu.VMEM((BLKV,), jnp.float32),
            [pltpu.VMEM((16,), jnp.float32) for _ in range(8)],
            pltpu.SemaphoreType.DMA,
            pltpu.SemaphoreType.DMA,
            pltpu.SemaphoreType.DMA,
        ],
    )(_body)
    return f(src_flat, flow_flat)


def kernel(source, flow_field):
    src_flat = source.reshape(N)
    flow_flat = flow_field.reshape(3 * N)
    out = _run(src_flat, flow_flat)
    return out.reshape(source.shape)
